# baseline (speedup 1.0000x reference)
import functools

import jax
import jax.numpy as jnp
from jax import lax
from jax.experimental import pallas as pl
from jax.experimental.pallas import tpu as pltpu
from jax.experimental.pallas import tpu_sc as plsc

_NBUF = 8
_CPLANES = 2


def _sc_gather_body(ids_hbm, w_hbm, b_hbm, wg_hbm, bg_hbm,
                    idx_v, wv, bv, out_w, out_b):
    single = jnp.logical_and(lax.axis_index("c") == 0, lax.axis_index("s") == 0)
    C = out_w.shape[0]
    num_sat = w_hbm.shape[0] // C

    @pl.when(single)
    def _():
        pltpu.sync_copy(ids_hbm, idx_v)
        pltpu.sync_copy(w_hbm, wv)
        pltpu.sync_copy(b_hbm, bv)
        v = idx_v[...]
        valid = jnp.logical_and(v >= 0, v < num_sat)
        base = jnp.where(valid, v, 0) * C
        for c in range(C):
            wg = plsc.load_gather(wv, [base + c])
            bg = plsc.load_gather(bv, [base + c])
            out_w[c] = jnp.where(valid, wg, jnp.float32(1.0))
            out_b[c] = jnp.where(valid, bg, jnp.float32(0.0))
        pltpu.sync_copy(out_w, wg_hbm)
        pltpu.sync_copy(out_b, bg_hbm)


def _sc_gather(ids, wflat, bflat, C):
    BN = ids.shape[0]
    S = wflat.shape[0]
    mesh = plsc.VectorSubcoreMesh(core_axis_name="c", subcore_axis_name="s")
    fn = functools.partial(
        pl.kernel,
        out_type=[jax.ShapeDtypeStruct((C, BN), jnp.float32)] * 2,
        mesh=mesh,
        scratch_types=[
            pltpu.VMEM((BN,), jnp.int32),
            pltpu.VMEM((S,), jnp.float32),
            pltpu.VMEM((S,), jnp.float32),
            pltpu.VMEM((C, BN), jnp.float32),
            pltpu.VMEM((C, BN), jnp.float32),
        ],
        compiler_params=pltpu.CompilerParams(needs_layout_passes=False),
    )(_sc_gather_body)
    return fn(ids, wflat, bflat)


def _affine_body(wg_ref, bg_ref, x_hbm, o_hbm, xbuf, obuf, in_sems, out_sems):
    n_chunks = x_hbm.shape[0]
    C = wg_ref.shape[0]
    H = x_hbm.shape[1] // _CPLANES
    per_sample = C // _CPLANES

    def in_dma(k, slot):
        return pltpu.make_async_copy(x_hbm.at[k], xbuf.at[slot], in_sems.at[slot])

    def out_dma(k, slot):
        return pltpu.make_async_copy(obuf.at[slot], o_hbm.at[k], out_sems.at[slot])

    for k in range(_NBUF):
        in_dma(k, k).start()

    def step(k, carry):
        slot = lax.rem(k, _NBUF)
        in_dma(k, slot).wait()

        @pl.when(k >= _NBUF)
        def _():
            out_dma(k - _NBUF, slot).wait()

        i = k // per_sample
        c0 = lax.rem(k, per_sample) * _CPLANES
        for j in range(_CPLANES):
            w = wg_ref[c0 + j, i]
            b = bg_ref[c0 + j, i]
            obuf[slot, pl.ds(j * H, H), :] = xbuf[slot, pl.ds(j * H, H), :] * w + b
        out_dma(k, slot).start()

        @pl.when(k + _NBUF < n_chunks)
        def _():
            in_dma(k + _NBUF, slot).start()

        return carry

    lax.fori_loop(0, n_chunks, step, 0)

    for k in range(n_chunks - _NBUF, n_chunks):
        out_dma(k, k % _NBUF).wait()


def kernel(x, satellite_ids, weight, bias):
    B, N, C, H, W = x.shape
    S = weight.shape[0]
    n_chunks = B * N * C // _CPLANES
    xr = x.reshape(n_chunks, _CPLANES * H, W)
    ids = satellite_ids.reshape(-1).astype(jnp.int32)
    w_g, b_g = _sc_gather(ids, weight.reshape(S * C), bias.reshape(S * C), C)
    grid_spec = pltpu.PrefetchScalarGridSpec(
        num_scalar_prefetch=2,
        grid=(1,),
        in_specs=[pl.BlockSpec(memory_space=pltpu.MemorySpace.HBM)],
        out_specs=pl.BlockSpec(memory_space=pltpu.MemorySpace.HBM),
        scratch_shapes=[
            pltpu.VMEM((_NBUF, _CPLANES * H, W), jnp.float32),
            pltpu.VMEM((_NBUF, _CPLANES * H, W), jnp.float32),
            pltpu.SemaphoreType.DMA((_NBUF,)),
            pltpu.SemaphoreType.DMA((_NBUF,)),
        ],
    )
    out = pl.pallas_call(
        _affine_body,
        grid_spec=grid_spec,
        out_shape=jax.ShapeDtypeStruct((n_chunks, _CPLANES * H, W), x.dtype),
    )(w_g, b_g, xr)
    return out.reshape(B, N, C, H, W)


# speedup vs baseline: 1.3912x; 1.3912x over previous
import jax
import jax.numpy as jnp
from jax.experimental import pallas as pl
from jax.experimental.pallas import tpu as pltpu

_NBUF = 4
_CPLANES = 4


def _affine_body(ids_ref, w_ref, b_ref, x_hbm, o_hbm, xbuf, obuf, in_sems, out_sems):
    n_chunks = x_hbm.shape[0]
    C = w_ref.shape[1]
    num_sat = w_ref.shape[0]
    H = x_hbm.shape[1] // _CPLANES
    per_sample = C // _CPLANES

    def in_dma(k, slot):
        return pltpu.make_async_copy(x_hbm.at[k], xbuf.at[slot], in_sems.at[slot])

    def out_dma(k, slot):
        return pltpu.make_async_copy(obuf.at[slot], o_hbm.at[k], out_sems.at[slot])

    for k in range(_NBUF):
        in_dma(k, k).start()

    def step(k, carry):
        slot = jax.lax.rem(k, _NBUF)
        in_dma(k, slot).wait()

        @pl.when(k >= _NBUF)
        def _():
            out_dma(k - _NBUF, slot).wait()

        sid = ids_ref[k // per_sample]
        valid = jnp.logical_and(sid >= 0, sid < num_sat)
        s = jnp.where(valid, sid, 0)
        c0 = jax.lax.rem(k, per_sample) * _CPLANES
        for j in range(_CPLANES):
            w = jnp.where(valid, w_ref[s, c0 + j], jnp.float32(1.0))
            b = jnp.where(valid, b_ref[s, c0 + j], jnp.float32(0.0))
            obuf[slot, pl.ds(j * H, H), :] = xbuf[slot, pl.ds(j * H, H), :] * w + b
        out_dma(k, slot).start()

        @pl.when(k + _NBUF < n_chunks)
        def _():
            in_dma(k + _NBUF, slot).start()

        return carry

    jax.lax.fori_loop(0, n_chunks, step, 0)

    for k in range(n_chunks - _NBUF, n_chunks):
        out_dma(k, k % _NBUF).wait()


def kernel(x, satellite_ids, weight, bias):
    B, N, C, H, W = x.shape
    S = weight.shape[0]
    n_chunks = B * N * C // _CPLANES
    xr = x.reshape(n_chunks, _CPLANES * H, W)
    ids = satellite_ids.reshape(-1).astype(jnp.int32)
    w2 = weight.reshape(S, C)
    b2 = bias.reshape(S, C)
    grid_spec = pltpu.PrefetchScalarGridSpec(
        num_scalar_prefetch=3,
        grid=(1,),
        in_specs=[pl.BlockSpec(memory_space=pltpu.MemorySpace.HBM)],
        out_specs=pl.BlockSpec(memory_space=pltpu.MemorySpace.HBM),
        scratch_shapes=[
            pltpu.VMEM((_NBUF, _CPLANES * H, W), jnp.float32),
            pltpu.VMEM((_NBUF, _CPLANES * H, W), jnp.float32),
            pltpu.SemaphoreType.DMA((_NBUF,)),
            pltpu.SemaphoreType.DMA((_NBUF,)),
        ],
    )
    out = pl.pallas_call(
        _affine_body,
        grid_spec=grid_spec,
        out_shape=jax.ShapeDtypeStruct((n_chunks, _CPLANES * H, W), x.dtype),
    )(ids, w2, b2, xr)
    return out.reshape(B, N, C, H, W)


# speedup vs baseline: 1.3966x; 1.0039x over previous
import jax
import jax.numpy as jnp
from jax.experimental import pallas as pl
from jax.experimental.pallas import tpu as pltpu

_NBUF = 6
_CPLANES = 4


def _affine_body(ids_ref, w_ref, b_ref, x_hbm, o_hbm, xbuf, obuf, in_sems, out_sems):
    n_chunks = x_hbm.shape[0]
    C = w_ref.shape[1]
    num_sat = w_ref.shape[0]
    H = x_hbm.shape[1] // _CPLANES
    per_sample = C // _CPLANES

    def in_dma(k, slot):
        return pltpu.make_async_copy(x_hbm.at[k], xbuf.at[slot], in_sems.at[slot])

    def out_dma(k, slot):
        return pltpu.make_async_copy(obuf.at[slot], o_hbm.at[k], out_sems.at[slot])

    for k in range(_NBUF):
        in_dma(k, k).start()

    def step(k, carry):
        slot = jax.lax.rem(k, _NBUF)
        in_dma(k, slot).wait()

        @pl.when(k >= _NBUF)
        def _():
            out_dma(k - _NBUF, slot).wait()

        sid = ids_ref[k // per_sample]
        valid = jnp.logical_and(sid >= 0, sid < num_sat)
        s = jnp.where(valid, sid, 0)
        c0 = jax.lax.rem(k, per_sample) * _CPLANES
        for j in range(_CPLANES):
            w = jnp.where(valid, w_ref[s, c0 + j], jnp.float32(1.0))
            b = jnp.where(valid, b_ref[s, c0 + j], jnp.float32(0.0))
            obuf[slot, pl.ds(j * H, H), :] = xbuf[slot, pl.ds(j * H, H), :] * w + b
        out_dma(k, slot).start()

        @pl.when(k + _NBUF < n_chunks)
        def _():
            in_dma(k + _NBUF, slot).start()

        return carry

    jax.lax.fori_loop(0, n_chunks, step, 0)

    for k in range(n_chunks - _NBUF, n_chunks):
        out_dma(k, k % _NBUF).wait()


def kernel(x, satellite_ids, weight, bias):
    B, N, C, H, W = x.shape
    S = weight.shape[0]
    n_chunks = B * N * C // _CPLANES
    xr = x.reshape(n_chunks, _CPLANES * H, W)
    ids = satellite_ids.reshape(-1).astype(jnp.int32)
    w2 = weight.reshape(S, C)
    b2 = bias.reshape(S, C)
    grid_spec = pltpu.PrefetchScalarGridSpec(
        num_scalar_prefetch=3,
        grid=(1,),
        in_specs=[pl.BlockSpec(memory_space=pltpu.MemorySpace.HBM)],
        out_specs=pl.BlockSpec(memory_space=pltpu.MemorySpace.HBM),
        scratch_shapes=[
            pltpu.VMEM((_NBUF, _CPLANES * H, W), jnp.float32),
            pltpu.VMEM((_NBUF, _CPLANES * H, W), jnp.float32),
            pltpu.SemaphoreType.DMA((_NBUF,)),
            pltpu.SemaphoreType.DMA((_NBUF,)),
        ],
    )
    out = pl.pallas_call(
        _affine_body,
        grid_spec=grid_spec,
        out_shape=jax.ShapeDtypeStruct((n_chunks, _CPLANES * H, W), x.dtype),
    )(ids, w2, b2, xr)
    return out.reshape(B, N, C, H, W)
